# baseline (speedup 1.0000x reference)
import jax
import jax.numpy as jnp
from jax import lax
from jax.experimental import pallas as pl
from jax.experimental.pallas import tpu as pltpu
from jax.experimental.pallas import tpu_sc as plsc

_N = 320000
_D = 128
_MAX_LEN = 240

_NUM_WORKERS = 32
_C = 200
_G1 = 128
_G2 = _C - _G1
_NBUF = 4
_PF = 2

_N_SC = 192000
_N_TC = _N - _N_SC
_TCB = 512


def _fuse_table_kernel(emb_ref, w_ref, b_ref, out_ref):
    out_ref[:, :] = (
        lax.dot_general(
            emb_ref[:, :], w_ref[:, :],
            dimension_numbers=(((1,), (1,)), ((), ())),
            preferred_element_type=jnp.float32,
        )
        + b_ref[:, :]
    )


def _make_sc_body(rows_per_worker):
    niter = rows_per_worker // _C
    nfull = (niter // _NBUF) * _NBUF

    def _sc_body(x_hbm, t_hbm, tab_hbm, out_hbm, tab_sh, idx_v, *bufs):
        xb = bufs[0:_NBUF]
        x_sem = bufs[_NBUF:2 * _NBUF]
        g_sem = bufs[2 * _NBUF:3 * _NBUF]
        o_sem = bufs[3 * _NBUF:4 * _NBUF]

        wid = lax.axis_index("s") * 2 + lax.axis_index("c")
        row_base = wid * rows_per_worker

        pltpu.sync_copy(t_hbm.at[pl.ds(row_base, rows_per_worker)], idx_v)

        @pl.when(lax.axis_index("s") == 0)
        def _stage_table():
            pltpu.sync_copy(tab_hbm, tab_sh)

        plsc.subcore_barrier()

        def x_slice(c):
            return x_hbm.at[pl.ds(row_base + c * _C, _C), :]

        def out_slice(c):
            return out_hbm.at[pl.ds(row_base + c * _C, _C), :]

        def gadd(c, b):
            pltpu.async_copy(
                tab_sh.at[idx_v.at[pl.ds(c * _C, _G1)]],
                xb[b].at[pl.ds(0, _G1), :], g_sem[b], add=True,
            )
            pltpu.async_copy(
                tab_sh.at[idx_v.at[pl.ds(c * _C + _G1, _G2)]],
                xb[b].at[pl.ds(_G1, _G2), :], g_sem[b], add=True,
            )

        def wait_gadd(c, b):
            pltpu.make_async_copy(
                tab_sh.at[idx_v.at[pl.ds(c * _C, _G1)]],
                xb[b].at[pl.ds(0, _G1), :], g_sem[b],
            ).wait()
            pltpu.make_async_copy(
                tab_sh.at[idx_v.at[pl.ds(c * _C + _G1, _G2)]],
                xb[b].at[pl.ds(_G1, _G2), :], g_sem[b],
            ).wait()

        def chunk_body(c, bi, tail):
            bn = (bi + 1) % _NBUF
            pf = c + _PF
            bpf = (bi + _PF) % _NBUF

            if not tail:
                @pl.when(pf < niter)
                def _prefetch():
                    @pl.when(pf >= _NBUF)
                    def _drain():
                        pltpu.make_async_copy(
                            xb[bpf], out_slice(pf - _NBUF), o_sem[bpf]
                        ).wait()

                    pltpu.async_copy(x_slice(pf), xb[bpf], x_sem[bpf])

            if not (tail and c + 1 >= niter):
                @pl.when(c + 1 < niter)
                def _next_gadd():
                    pltpu.make_async_copy(
                        x_slice(c + 1), xb[bn], x_sem[bn]
                    ).wait()
                    gadd(c + 1, bn)

            wait_gadd(c, bi)
            pltpu.async_copy(xb[bi], out_slice(c), o_sem[bi])

        for i in range(_PF):
            pltpu.async_copy(x_slice(i), xb[i], x_sem[i])
        pltpu.make_async_copy(x_slice(0), xb[0], x_sem[0]).wait()
        gadd(0, 0)

        def outer(k, carry):
            for bi in range(_NBUF):
                chunk_body(k * _NBUF + bi, bi, tail=False)
            return carry

        lax.fori_loop(0, nfull // _NBUF, outer, 0, unroll=False)

        for c in range(nfull, niter):
            chunk_body(c, c % _NBUF, tail=True)

        for c_last in range(niter - _NBUF, niter):
            bi = c_last % _NBUF
            pltpu.make_async_copy(xb[bi], out_slice(c_last), o_sem[bi]).wait()

    return _sc_body


def _sc_lookup(x, t, fused_table):
    n = x.shape[0]
    rows_per_worker = n // _NUM_WORKERS
    mesh = plsc.VectorSubcoreMesh(core_axis_name="c", subcore_axis_name="s")
    scratch = (
        [pltpu.VMEM_SHARED((_MAX_LEN, _D), jnp.float32)]
        + [pltpu.VMEM((rows_per_worker,), jnp.int32)]
        + [pltpu.VMEM((_C, _D), jnp.float32) for _ in range(_NBUF)]
        + [pltpu.SemaphoreType.DMA for _ in range(3 * _NBUF)]
    )
    return pl.kernel(
        _make_sc_body(rows_per_worker),
        out_type=jax.ShapeDtypeStruct((n, _D), jnp.float32),
        mesh=mesh,
        scratch_types=scratch,
        compiler_params=pltpu.CompilerParams(needs_layout_passes=False),
    )(x, t, fused_table)


def _tc_lookup_kernel(x_ref, t_ref, tab_ref, out_ref):
    tab = tab_ref[:, :]
    tab_hi = tab.astype(jnp.bfloat16)
    tab_lo = (tab - tab_hi.astype(jnp.float32)).astype(jnp.bfloat16)
    tv = t_ref[0, 0, :]
    oh = (
        lax.broadcasted_iota(jnp.int32, (_TCB, _MAX_LEN), 1)
        == tv[:, None]
    ).astype(jnp.bfloat16)
    dn = (((1,), (0,)), ((), ()))
    y = lax.dot_general(
        oh, tab_hi, dimension_numbers=dn,
        preferred_element_type=jnp.float32,
    ) + lax.dot_general(
        oh, tab_lo, dimension_numbers=dn,
        preferred_element_type=jnp.float32,
    )
    out_ref[:, :] = x_ref[:, :] + y


def _tc_lookup(x, t, fused_table):
    nblk = x.shape[0] // _TCB
    t3 = t.reshape(nblk, 1, _TCB)
    return pl.pallas_call(
        _tc_lookup_kernel,
        grid=(nblk,),
        in_specs=[
            pl.BlockSpec((_TCB, _D), lambda i: (i, 0)),
            pl.BlockSpec((1, 1, _TCB), lambda i: (i, 0, 0)),
            pl.BlockSpec((_MAX_LEN, _D), lambda i: (0, 0)),
        ],
        out_specs=pl.BlockSpec((_TCB, _D), lambda i: (i, 0)),
        out_shape=jax.ShapeDtypeStruct((x.shape[0], _D), jnp.float32),
    )(x, t3, fused_table)


def kernel(x, t, emb_table, W, b):
    fused_table = pl.pallas_call(
        _fuse_table_kernel,
        out_shape=jax.ShapeDtypeStruct((_MAX_LEN, _D), jnp.float32),
    )(emb_table, W, b.reshape(1, _D))

    out_sc = _sc_lookup(x[:_N_SC], t[:_N_SC], fused_table)
    out_tc = _tc_lookup(x[_N_SC:], t[_N_SC:], fused_table)
    return jnp.concatenate([out_sc, out_tc], axis=0)


# speedup vs baseline: 3.4603x; 3.4603x over previous
import jax
import jax.numpy as jnp
from jax import lax
from jax.experimental import pallas as pl
from jax.experimental.pallas import tpu as pltpu
from jax.experimental.pallas import tpu_sc as plsc

_N = 320000
_D = 128
_MAX_LEN = 240

_NUM_WORKERS = 32
_C = 80
_G1 = min(_C, 128)
_G2 = _C - _G1
_NBUF = 5
_PF = 3

_N_SC = 192000
_N_TC = _N - _N_SC
_TCB = 512


def _fuse_table_kernel(emb_ref, w_ref, b_ref, out_ref):
    out_ref[:, :] = (
        lax.dot_general(
            emb_ref[:, :], w_ref[:, :],
            dimension_numbers=(((1,), (1,)), ((), ())),
            preferred_element_type=jnp.float32,
        )
        + b_ref[:, :]
    )


def _make_sc_body(rows_per_worker):
    niter = rows_per_worker // _C
    nfull = (niter // _NBUF) * _NBUF

    def _sc_body(x_hbm, t_hbm, tab_hbm, out_hbm, tab_sh, idx_v, *bufs):
        xb = bufs[0:_NBUF]
        x_sem = bufs[_NBUF:2 * _NBUF]
        g_sem = bufs[2 * _NBUF:3 * _NBUF]
        o_sem = bufs[3 * _NBUF:4 * _NBUF]

        wid = lax.axis_index("s") * 2 + lax.axis_index("c")
        row_base = wid * rows_per_worker

        pltpu.sync_copy(t_hbm.at[pl.ds(row_base, rows_per_worker)], idx_v)

        @pl.when(lax.axis_index("s") == 0)
        def _stage_table():
            pltpu.sync_copy(tab_hbm, tab_sh)

        plsc.subcore_barrier()

        def x_slice(c):
            return x_hbm.at[pl.ds(row_base + c * _C, _C), :]

        def out_slice(c):
            return out_hbm.at[pl.ds(row_base + c * _C, _C), :]

        def gadd(c, b):
            pltpu.async_copy(
                tab_sh.at[idx_v.at[pl.ds(c * _C, _G1)]],
                xb[b].at[pl.ds(0, _G1), :], g_sem[b], add=True,
            )
            if _G2:
                pltpu.async_copy(
                    tab_sh.at[idx_v.at[pl.ds(c * _C + _G1, _G2)]],
                    xb[b].at[pl.ds(_G1, _G2), :], g_sem[b], add=True,
                )

        def wait_gadd(c, b):
            pltpu.make_async_copy(
                tab_sh.at[idx_v.at[pl.ds(c * _C, _G1)]],
                xb[b].at[pl.ds(0, _G1), :], g_sem[b],
            ).wait()
            if _G2:
                pltpu.make_async_copy(
                    tab_sh.at[idx_v.at[pl.ds(c * _C + _G1, _G2)]],
                    xb[b].at[pl.ds(_G1, _G2), :], g_sem[b],
                ).wait()

        def chunk_body(c, bi, tail):
            bn = (bi + 1) % _NBUF
            pf = c + _PF
            bpf = (bi + _PF) % _NBUF

            if not tail:
                @pl.when(pf < niter)
                def _prefetch():
                    @pl.when(pf >= _NBUF)
                    def _drain():
                        pltpu.make_async_copy(
                            xb[bpf], out_slice(pf - _NBUF), o_sem[bpf]
                        ).wait()

                    pltpu.async_copy(x_slice(pf), xb[bpf], x_sem[bpf])

            if not (tail and c + 1 >= niter):
                @pl.when(c + 1 < niter)
                def _next_gadd():
                    pltpu.make_async_copy(
                        x_slice(c + 1), xb[bn], x_sem[bn]
                    ).wait()
                    gadd(c + 1, bn)

            wait_gadd(c, bi)
            pltpu.async_copy(xb[bi], out_slice(c), o_sem[bi])

        for i in range(_PF):
            pltpu.async_copy(x_slice(i), xb[i], x_sem[i])
        pltpu.make_async_copy(x_slice(0), xb[0], x_sem[0]).wait()
        gadd(0, 0)

        def outer(k, carry):
            for bi in range(_NBUF):
                chunk_body(k * _NBUF + bi, bi, tail=False)
            return carry

        lax.fori_loop(0, nfull // _NBUF, outer, 0, unroll=False)

        for c in range(nfull, niter):
            chunk_body(c, c % _NBUF, tail=True)

        for c_last in range(niter - _NBUF, niter):
            bi = c_last % _NBUF
            pltpu.make_async_copy(xb[bi], out_slice(c_last), o_sem[bi]).wait()

    return _sc_body


def _sc_lookup(x, t, fused_table):
    n = x.shape[0]
    rows_per_worker = n // _NUM_WORKERS
    mesh = plsc.VectorSubcoreMesh(core_axis_name="c", subcore_axis_name="s")
    scratch = (
        [pltpu.VMEM_SHARED((_MAX_LEN, _D), jnp.float32)]
        + [pltpu.VMEM((rows_per_worker,), jnp.int32)]
        + [pltpu.VMEM((_C, _D), jnp.float32) for _ in range(_NBUF)]
        + [pltpu.SemaphoreType.DMA for _ in range(3 * _NBUF)]
    )
    return pl.kernel(
        _make_sc_body(rows_per_worker),
        out_type=jax.ShapeDtypeStruct((n, _D), jnp.float32),
        mesh=mesh,
        scratch_types=scratch,
        compiler_params=pltpu.CompilerParams(needs_layout_passes=False),
    )(x, t, fused_table)


def _tc_lookup_kernel(x_ref, t_ref, tab_ref, out_ref):
    tab = tab_ref[:, :]
    tab_hi = tab.astype(jnp.bfloat16)
    tab_lo = (tab - tab_hi.astype(jnp.float32)).astype(jnp.bfloat16)
    tv = t_ref[0, 0, :]
    oh = (
        lax.broadcasted_iota(jnp.int32, (_TCB, _MAX_LEN), 1)
        == tv[:, None]
    ).astype(jnp.bfloat16)
    dn = (((1,), (0,)), ((), ()))
    y = lax.dot_general(
        oh, tab_hi, dimension_numbers=dn,
        preferred_element_type=jnp.float32,
    ) + lax.dot_general(
        oh, tab_lo, dimension_numbers=dn,
        preferred_element_type=jnp.float32,
    )
    out_ref[:, :] = x_ref[:, :] + y


def _tc_lookup(x, t, fused_table):
    nblk = x.shape[0] // _TCB
    t3 = t.reshape(nblk, 1, _TCB)
    return pl.pallas_call(
        _tc_lookup_kernel,
        grid=(nblk,),
        in_specs=[
            pl.BlockSpec((_TCB, _D), lambda i: (i, 0)),
            pl.BlockSpec((1, 1, _TCB), lambda i: (i, 0, 0)),
            pl.BlockSpec((_MAX_LEN, _D), lambda i: (0, 0)),
        ],
        out_specs=pl.BlockSpec((_TCB, _D), lambda i: (i, 0)),
        out_shape=jax.ShapeDtypeStruct((x.shape[0], _D), jnp.float32),
    )(x, t3, fused_table)


def kernel(x, t, emb_table, W, b):
    fused_table = pl.pallas_call(
        _fuse_table_kernel,
        out_shape=jax.ShapeDtypeStruct((_MAX_LEN, _D), jnp.float32),
    )(emb_table, W, b.reshape(1, _D))

    return _sc_lookup(x, t, fused_table)


# speedup vs baseline: 3.4858x; 1.0074x over previous
import jax
import jax.numpy as jnp
from jax import lax
from jax.experimental import pallas as pl
from jax.experimental.pallas import tpu as pltpu
from jax.experimental.pallas import tpu_sc as plsc

_N = 320000
_D = 128
_MAX_LEN = 240

_NUM_WORKERS = 32
_C = 80
_G1 = min(_C, 128)
_G2 = _C - _G1
_NBUF = 10
_PF = 5

_N_SC = 192000
_N_TC = _N - _N_SC
_TCB = 512


def _fuse_table_kernel(emb_ref, w_ref, b_ref, out_ref):
    out_ref[:, :] = (
        lax.dot_general(
            emb_ref[:, :], w_ref[:, :],
            dimension_numbers=(((1,), (1,)), ((), ())),
            preferred_element_type=jnp.float32,
        )
        + b_ref[:, :]
    )


def _make_sc_body(rows_per_worker):
    niter = rows_per_worker // _C
    nfull = (niter // _NBUF) * _NBUF

    def _sc_body(x_hbm, t_hbm, tab_hbm, out_hbm, tab_sh, idx_v, *bufs):
        xb = bufs[0:_NBUF]
        x_sem = bufs[_NBUF:2 * _NBUF]
        g_sem = bufs[2 * _NBUF:3 * _NBUF]
        o_sem = bufs[3 * _NBUF:4 * _NBUF]

        wid = lax.axis_index("s") * 2 + lax.axis_index("c")
        row_base = wid * rows_per_worker

        pltpu.sync_copy(t_hbm.at[pl.ds(row_base, rows_per_worker)], idx_v)

        @pl.when(lax.axis_index("s") == 0)
        def _stage_table():
            pltpu.sync_copy(tab_hbm, tab_sh)

        plsc.subcore_barrier()

        def x_slice(c):
            return x_hbm.at[pl.ds(row_base + c * _C, _C), :]

        def out_slice(c):
            return out_hbm.at[pl.ds(row_base + c * _C, _C), :]

        def gadd(c, b):
            pltpu.async_copy(
                tab_sh.at[idx_v.at[pl.ds(c * _C, _G1)]],
                xb[b].at[pl.ds(0, _G1), :], g_sem[b], add=True,
            )
            if _G2:
                pltpu.async_copy(
                    tab_sh.at[idx_v.at[pl.ds(c * _C + _G1, _G2)]],
                    xb[b].at[pl.ds(_G1, _G2), :], g_sem[b], add=True,
                )

        def wait_gadd(c, b):
            pltpu.make_async_copy(
                tab_sh.at[idx_v.at[pl.ds(c * _C, _G1)]],
                xb[b].at[pl.ds(0, _G1), :], g_sem[b],
            ).wait()
            if _G2:
                pltpu.make_async_copy(
                    tab_sh.at[idx_v.at[pl.ds(c * _C + _G1, _G2)]],
                    xb[b].at[pl.ds(_G1, _G2), :], g_sem[b],
                ).wait()

        def chunk_body(c, bi, tail):
            bn = (bi + 1) % _NBUF
            pf = c + _PF
            bpf = (bi + _PF) % _NBUF

            if not tail:
                @pl.when(pf < niter)
                def _prefetch():
                    @pl.when(pf >= _NBUF)
                    def _drain():
                        pltpu.make_async_copy(
                            xb[bpf], out_slice(pf - _NBUF), o_sem[bpf]
                        ).wait()

                    pltpu.async_copy(x_slice(pf), xb[bpf], x_sem[bpf])

            if not (tail and c + 1 >= niter):
                @pl.when(c + 1 < niter)
                def _next_gadd():
                    pltpu.make_async_copy(
                        x_slice(c + 1), xb[bn], x_sem[bn]
                    ).wait()
                    gadd(c + 1, bn)

            wait_gadd(c, bi)
            pltpu.async_copy(xb[bi], out_slice(c), o_sem[bi])

        for i in range(_PF):
            pltpu.async_copy(x_slice(i), xb[i], x_sem[i])
        pltpu.make_async_copy(x_slice(0), xb[0], x_sem[0]).wait()
        gadd(0, 0)

        def outer(k, carry):
            for bi in range(_NBUF):
                chunk_body(k * _NBUF + bi, bi, tail=False)
            return carry

        lax.fori_loop(0, nfull // _NBUF, outer, 0, unroll=False)

        for c in range(nfull, niter):
            chunk_body(c, c % _NBUF, tail=True)

        for c_last in range(niter - _NBUF, niter):
            bi = c_last % _NBUF
            pltpu.make_async_copy(xb[bi], out_slice(c_last), o_sem[bi]).wait()

    return _sc_body


def _sc_lookup(x, t, fused_table):
    n = x.shape[0]
    rows_per_worker = n // _NUM_WORKERS
    mesh = plsc.VectorSubcoreMesh(core_axis_name="c", subcore_axis_name="s")
    scratch = (
        [pltpu.VMEM_SHARED((_MAX_LEN, _D), jnp.float32)]
        + [pltpu.VMEM((rows_per_worker,), jnp.int32)]
        + [pltpu.VMEM((_C, _D), jnp.float32) for _ in range(_NBUF)]
        + [pltpu.SemaphoreType.DMA for _ in range(3 * _NBUF)]
    )
    return pl.kernel(
        _make_sc_body(rows_per_worker),
        out_type=jax.ShapeDtypeStruct((n, _D), jnp.float32),
        mesh=mesh,
        scratch_types=scratch,
        compiler_params=pltpu.CompilerParams(needs_layout_passes=False),
    )(x, t, fused_table)


def _tc_lookup_kernel(x_ref, t_ref, tab_ref, out_ref):
    tab = tab_ref[:, :]
    tab_hi = tab.astype(jnp.bfloat16)
    tab_lo = (tab - tab_hi.astype(jnp.float32)).astype(jnp.bfloat16)
    tv = t_ref[0, 0, :]
    oh = (
        lax.broadcasted_iota(jnp.int32, (_TCB, _MAX_LEN), 1)
        == tv[:, None]
    ).astype(jnp.bfloat16)
    dn = (((1,), (0,)), ((), ()))
    y = lax.dot_general(
        oh, tab_hi, dimension_numbers=dn,
        preferred_element_type=jnp.float32,
    ) + lax.dot_general(
        oh, tab_lo, dimension_numbers=dn,
        preferred_element_type=jnp.float32,
    )
    out_ref[:, :] = x_ref[:, :] + y


def _tc_lookup(x, t, fused_table):
    nblk = x.shape[0] // _TCB
    t3 = t.reshape(nblk, 1, _TCB)
    return pl.pallas_call(
        _tc_lookup_kernel,
        grid=(nblk,),
        in_specs=[
            pl.BlockSpec((_TCB, _D), lambda i: (i, 0)),
            pl.BlockSpec((1, 1, _TCB), lambda i: (i, 0, 0)),
            pl.BlockSpec((_MAX_LEN, _D), lambda i: (0, 0)),
        ],
        out_specs=pl.BlockSpec((_TCB, _D), lambda i: (i, 0)),
        out_shape=jax.ShapeDtypeStruct((x.shape[0], _D), jnp.float32),
    )(x, t3, fused_table)


def kernel(x, t, emb_table, W, b):
    fused_table = pl.pallas_call(
        _fuse_table_kernel,
        out_shape=jax.ShapeDtypeStruct((_MAX_LEN, _D), jnp.float32),
    )(emb_table, W, b.reshape(1, _D))

    return _sc_lookup(x, t, fused_table)
